# baseline (speedup 1.0000x reference)
import jax
import jax.numpy as jnp
from jax import lax
from jax.experimental import pallas as pl
from jax.experimental.pallas import tpu as pltpu
from jax.experimental.pallas import tpu_sc as plsc

B, N, K, H = 8, 64, 128, 768
V = 100000
L = K + 1
CLS_SEG = 1
EPS = 1e-12

NC, NS = 2, 16
NW = NC * NS
PAIRS = B * N
PPW = PAIRS // NW
WPB = N // PPW
TPW = PPW * L
LP = 136
CL = 16
NCH = K // CL
NU = NCH * PPW
NSL = H // 16
UBYTES = CL * H * 4


def _rsqrt(x):
    xi = lax.bitcast_convert_type(x, jnp.int32)
    yi = jnp.full((16,), 0x5F3759DF, jnp.int32) - (xi >> 1)
    y = lax.bitcast_convert_type(yi, jnp.float32)
    for _ in range(3):
        y = y * (1.5 - 0.5 * x * y * y)
    return y


_GATHER_DNUMS = lax.GatherDimensionNumbers(
    offset_dims=(), collapsed_slice_dims=(0,), start_index_map=(0,))


def _shuffle(x, idx):
    return lax.gather(x, idx[:, None], dimension_numbers=_GATHER_DNUMS,
                      slice_sizes=(1,),
                      mode=lax.GatherScatterMode.PROMISE_IN_BOUNDS)


def _lane_sum(x):
    lanes = lax.iota(jnp.int32, 16)
    for s in (8, 4, 2, 1):
        x = x + _shuffle(x, lanes ^ s)
    return x


def _unit_norm(src, dst, pos_buf, seg_buf, sp, gam_buf, bet_buf, stats,
               rs_buf):
    lanes = lax.iota(jnp.int32, 16)

    accs = []
    for h in (0, 8):
        def j_body(j, acc, h=h):
            sl = pl.ds(pl.multiple_of(j * 16, 16), 16)
            seg_j = seg_buf[sp, sl]
            out = []
            for i in range(h, h + 8):
                x = src[i, sl] + pos_buf[i, sl] + seg_j
                dst[i, sl] = x
                out.append(acc[2 * (i - h)] + x)
                out.append(acc[2 * (i - h) + 1] + x * x)
            return tuple(out)

        acc0 = tuple(jnp.zeros((16,), jnp.float32) for _ in range(16))
        accs.append(lax.fori_loop(0, NSL, j_body, acc0))
    for i in range(16):
        acc = accs[i // 8]
        k = i % 8
        plsc.store_scatter(stats, [lanes + 17 * i], acc[2 * k])
        plsc.store_scatter(stats, [lanes + (272 + 17 * i)], acc[2 * k + 1])
    tot_s = jnp.zeros((16,), jnp.float32)
    tot_q = jnp.zeros((16,), jnp.float32)
    for c in range(16):
        tot_s = tot_s + plsc.load_gather(stats, [lanes * 17 + c])
        tot_q = tot_q + plsc.load_gather(stats, [lanes * 17 + (272 + c)])
    mean = tot_s * (1.0 / H)
    msq = tot_q * (1.0 / H)
    rinv = _rsqrt(msq - mean * mean + EPS)
    shift = -mean * rinv

    for h in (0, 8):
        ri = [jnp.full((16,), rinv[i], jnp.float32) for i in range(h, h + 8)]
        sh = [jnp.full((16,), shift[i], jnp.float32) for i in range(h, h + 8)]

        def j2_body(j, carry, h=h, ri=ri, sh=sh):
            sl = pl.ds(pl.multiple_of(j * 16, 16), 16)
            g = gam_buf[sl]
            b = bet_buf[sl]
            for i in range(h, h + 8):
                dst[i, sl] = (dst[i, sl] * ri[i - h] + sh[i - h]) * g + b
            return carry

        lax.fori_loop(0, NSL, j2_body, 0)


def _norm_row(src, si, pos_buf, pi, seg_buf, sp, gam_buf, bet_buf, dst, di):
    sum_v = jnp.zeros((16,), jnp.float32)
    sq_v = jnp.zeros((16,), jnp.float32)
    xs = []
    for j in range(NSL):
        sl = pl.ds(j * 16, 16)
        x = src[si, sl] + pos_buf[pi, sl] + seg_buf[sp, sl]
        xs.append(x)
        sum_v = sum_v + x
        sq_v = sq_v + x * x
    mean = _lane_sum(sum_v) * (1.0 / H)
    msq = _lane_sum(sq_v) * (1.0 / H)
    rinv = _rsqrt(msq - mean * mean + EPS)
    shift = -mean * rinv
    for j in range(NSL):
        sl = pl.ds(j * 16, 16)
        dst[di, sl] = (xs[j] * rinv + shift) * gam_buf[sl] + bet_buf[sl]


def _idx_off(u):
    return pl.multiple_of(lax.rem(u, PPW) * LP + (u // PPW) * CL, 8)


def _body(ids_ref, tail_ref, word_ref, pos_ref, seg_ref, gamma_ref, beta_ref,
          out_ref,
          idx0, ids_buf, gbuf0, gbuf1, obuf0, obuf1,
          pos_buf, seg_buf, gam_buf, bet_buf, stats, rs_buf,
          gsem0, gsem1, ssem0, ssem1):
    w = lax.axis_index("s") * NC + lax.axis_index("c")
    bb = w // WPB
    n0 = (w % WPB) * PPW
    gbuf = (gbuf0, gbuf1)
    obuf = (obuf0, obuf1)
    gsem = (gsem0, gsem1)
    ssem = (ssem0, ssem1)

    pltpu.sync_copy(gamma_ref, gam_buf)
    pltpu.sync_copy(beta_ref, bet_buf)
    pltpu.sync_copy(seg_ref.at[pl.ds(n0, PPW)], seg_buf)
    pltpu.sync_copy(pos_ref.at[pl.ds(0, CL)], pos_buf)
    pltpu.sync_copy(ids_ref.at[pl.ds(w * (PPW * LP), PPW * LP)], ids_buf)

    for s in range(2):
        pltpu.async_copy(word_ref.at[ids_buf.at[pl.ds(_idx_off(s), CL)]],
                         gbuf[s], gsem[s])
        pltpu.async_copy(seg_buf, out_ref.at[bb, n0 + s, pl.ds(0, CL)],
                         ssem[s])

    def step(k, carry):
        for s in range(2):
            u = k * 2 + s
            pltpu.make_async_copy(
                pos_ref.at[pl.ds(0, CL)], obuf[s], ssem[s]).wait()
            pltpu.make_async_copy(
                word_ref.at[ids_buf.at[pl.ds(_idx_off(u), CL)]],
                gbuf[s], gsem[s]).wait()

            if s == 0:
                @pl.when(lax.rem(u, PPW) == 0)
                def _():
                    lo = pl.multiple_of(u, PPW)
                    pltpu.sync_copy(pos_ref.at[pl.ds(lo, CL)], pos_buf)

            p = lax.rem(u, PPW)
            lo = pl.multiple_of(u - p, CL)
            _unit_norm(gbuf[s], obuf[s], pos_buf, seg_buf, p,
                       gam_buf, bet_buf, stats, rs_buf)

            @pl.when(u + 2 < NU)
            def _():
                pltpu.async_copy(
                    word_ref.at[ids_buf.at[pl.ds(_idx_off(u + 2), CL)]],
                    gbuf[s], gsem[s])

            pltpu.async_copy(obuf[s], out_ref.at[bb, n0 + p, pl.ds(lo, CL)],
                             ssem[s])
        return carry

    lax.fori_loop(0, NU // 2, step, 0)
    for s in range(2):
        pltpu.make_async_copy(pos_ref.at[pl.ds(0, CL)], obuf[s],
                              ssem[s]).wait()

    pltpu.sync_copy(tail_ref.at[pl.ds(w * PPW, PPW)], idx0)
    pltpu.async_copy(word_ref.at[idx0], gbuf0, gsem0).wait()
    pltpu.sync_copy(pos_ref.at[pl.ds(K, 8)], pos_buf.at[pl.ds(0, 8)])

    ktail = pl.multiple_of(jnp.int32(K) + 0 * n0, 8)

    def tail_row(p, carry):
        _norm_row(gbuf0, p, pos_buf, 0, seg_buf, p,
                  gam_buf, bet_buf, obuf0, 0)
        pltpu.sync_copy(obuf0.at[pl.ds(0, 8)],
                        out_ref.at[bb, n0 + p, pl.ds(ktail, 8)])
        return carry

    lax.fori_loop(0, PPW, tail_row, 0)


_sc_call = pl.kernel(
    _body,
    out_type=jax.ShapeDtypeStruct((B, N, L, H), jnp.float32),
    mesh=plsc.VectorSubcoreMesh(core_axis_name="c", subcore_axis_name="s"),
    compiler_params=pltpu.CompilerParams(needs_layout_passes=False),
    scratch_types=[
        pltpu.VMEM((CL,), jnp.int32),
        pltpu.VMEM((PPW * LP,), jnp.int32),
        pltpu.VMEM((CL, H), jnp.float32),
        pltpu.VMEM((CL, H), jnp.float32),
        pltpu.VMEM((CL, H), jnp.float32),
        pltpu.VMEM((CL, H), jnp.float32),
        pltpu.VMEM((CL, H), jnp.float32),
        pltpu.VMEM((PPW, H), jnp.float32),
        pltpu.VMEM((H,), jnp.float32),
        pltpu.VMEM((H,), jnp.float32),
        pltpu.VMEM((544,), jnp.float32),
        pltpu.VMEM((32,), jnp.float32),
        pltpu.SemaphoreType.DMA,
        pltpu.SemaphoreType.DMA,
        pltpu.SemaphoreType.DMA,
        pltpu.SemaphoreType.DMA,
    ],
)


@jax.jit
def kernel(input_ids, word_table, pos_table, seg_table, gamma, beta):
    ii = input_ids.astype(jnp.int32).reshape(PAIRS, K)
    ids1d = jnp.concatenate(
        [jnp.full((PAIRS, 1), CLS_SEG, jnp.int32), ii,
         jnp.zeros((PAIRS, LP - L), jnp.int32)], axis=1).reshape(-1)
    tail_ids = ii[:, K - 1]
    posp = jnp.pad(pos_table, ((0, 7), (0, 0)))
    return _sc_call(ids1d, tail_ids, word_table, posp, seg_table,
                    gamma, beta)


# speedup vs baseline: 1.0170x; 1.0170x over previous
import jax
import jax.numpy as jnp
from jax import lax
from jax.experimental import pallas as pl
from jax.experimental.pallas import tpu as pltpu
from jax.experimental.pallas import tpu_sc as plsc

B, N, K, H = 8, 64, 128, 768
V = 100000
L = K + 1
CLS_SEG = 1
EPS = 1e-12

NC, NS = 2, 16
NW = NC * NS
PAIRS = B * N
PPW = PAIRS // NW
WPB = N // PPW
TPW = PPW * L
LP = 136
CL = 16
NCH = K // CL
NU = NCH * PPW
NSL = H // 16
UBYTES = CL * H * 4


def _rsqrt(x):
    xi = lax.bitcast_convert_type(x, jnp.int32)
    yi = jnp.full((16,), 0x5F3759DF, jnp.int32) - (xi >> 1)
    y = lax.bitcast_convert_type(yi, jnp.float32)
    for _ in range(3):
        y = y * (1.5 - 0.5 * x * y * y)
    return y


_GATHER_DNUMS = lax.GatherDimensionNumbers(
    offset_dims=(), collapsed_slice_dims=(0,), start_index_map=(0,))


def _shuffle(x, idx):
    return lax.gather(x, idx[:, None], dimension_numbers=_GATHER_DNUMS,
                      slice_sizes=(1,),
                      mode=lax.GatherScatterMode.PROMISE_IN_BOUNDS)


def _lane_sum(x):
    lanes = lax.iota(jnp.int32, 16)
    for s in (8, 4, 2, 1):
        x = x + _shuffle(x, lanes ^ s)
    return x


def _unit_norm(src, dst, pos_buf, seg_buf, sp, gam_buf, bet_buf, stats,
               rs_buf):
    lanes = lax.iota(jnp.int32, 16)

    def j_body(j, acc):
        sl = pl.ds(pl.multiple_of(j * 16, 16), 16)
        seg_j = seg_buf[sp, sl]
        out = []
        for i in range(16):
            x = src[i, sl] + pos_buf[i, sl] + seg_j
            dst[i, sl] = x
            out.append(acc[2 * i] + x)
            out.append(acc[2 * i + 1] + x * x)
        return tuple(out)

    acc0 = tuple(jnp.zeros((16,), jnp.float32) for _ in range(32))
    acc = lax.fori_loop(0, NSL, j_body, acc0)
    for i in range(16):
        plsc.store_scatter(stats, [lanes + 17 * i], acc[2 * i])
        plsc.store_scatter(stats, [lanes + (272 + 17 * i)], acc[2 * i + 1])
    tot_s = jnp.zeros((16,), jnp.float32)
    tot_q = jnp.zeros((16,), jnp.float32)
    for c in range(16):
        tot_s = tot_s + plsc.load_gather(stats, [lanes * 17 + c])
        tot_q = tot_q + plsc.load_gather(stats, [lanes * 17 + (272 + c)])
    mean = tot_s * (1.0 / H)
    msq = tot_q * (1.0 / H)
    rinv = _rsqrt(msq - mean * mean + EPS)
    shift = -mean * rinv

    for h in (0, 8):
        ri = [jnp.full((16,), rinv[i], jnp.float32) for i in range(h, h + 8)]
        sh = [jnp.full((16,), shift[i], jnp.float32) for i in range(h, h + 8)]

        def j2_body(j, carry, h=h, ri=ri, sh=sh):
            sl = pl.ds(pl.multiple_of(j * 16, 16), 16)
            g = gam_buf[sl]
            b = bet_buf[sl]
            for i in range(h, h + 8):
                dst[i, sl] = (dst[i, sl] * ri[i - h] + sh[i - h]) * g + b
            return carry

        lax.fori_loop(0, NSL, j2_body, 0)


def _norm_row(src, si, pos_buf, pi, seg_buf, sp, gam_buf, bet_buf, dst, di):
    sum_v = jnp.zeros((16,), jnp.float32)
    sq_v = jnp.zeros((16,), jnp.float32)
    xs = []
    for j in range(NSL):
        sl = pl.ds(j * 16, 16)
        x = src[si, sl] + pos_buf[pi, sl] + seg_buf[sp, sl]
        xs.append(x)
        sum_v = sum_v + x
        sq_v = sq_v + x * x
    mean = _lane_sum(sum_v) * (1.0 / H)
    msq = _lane_sum(sq_v) * (1.0 / H)
    rinv = _rsqrt(msq - mean * mean + EPS)
    shift = -mean * rinv
    for j in range(NSL):
        sl = pl.ds(j * 16, 16)
        dst[di, sl] = (xs[j] * rinv + shift) * gam_buf[sl] + bet_buf[sl]


def _idx_off(u):
    return pl.multiple_of(lax.rem(u, PPW) * LP + (u // PPW) * CL, 8)


def _body(ids_ref, tail_ref, word_ref, pos_ref, seg_ref, gamma_ref, beta_ref,
          out_ref,
          idx0, ids_buf, gbuf0, gbuf1, obuf0, obuf1,
          pos_buf, seg_buf, gam_buf, bet_buf, stats, rs_buf,
          gsem0, gsem1, ssem0, ssem1):
    w = lax.axis_index("s") * NC + lax.axis_index("c")
    bb = w // WPB
    n0 = (w % WPB) * PPW
    gbuf = (gbuf0, gbuf1)
    obuf = (obuf0, obuf1)
    gsem = (gsem0, gsem1)
    ssem = (ssem0, ssem1)

    pltpu.sync_copy(gamma_ref, gam_buf)
    pltpu.sync_copy(beta_ref, bet_buf)
    pltpu.sync_copy(seg_ref.at[pl.ds(n0, PPW)], seg_buf)
    pltpu.sync_copy(pos_ref.at[pl.ds(0, CL)], pos_buf)
    pltpu.sync_copy(ids_ref.at[pl.ds(w * (PPW * LP), PPW * LP)], ids_buf)

    for s in range(2):
        pltpu.async_copy(word_ref.at[ids_buf.at[pl.ds(_idx_off(s), CL)]],
                         gbuf[s], gsem[s])
        pltpu.async_copy(seg_buf, out_ref.at[bb, n0 + s, pl.ds(0, CL)],
                         ssem[s])

    def step(k, carry):
        for s in range(2):
            u = k * 2 + s
            pltpu.make_async_copy(
                pos_ref.at[pl.ds(0, CL)], obuf[s], ssem[s]).wait()
            pltpu.make_async_copy(
                word_ref.at[ids_buf.at[pl.ds(_idx_off(u), CL)]],
                gbuf[s], gsem[s]).wait()

            if s == 0:
                @pl.when(lax.rem(u, PPW) == 0)
                def _():
                    lo = pl.multiple_of(u, PPW)
                    pltpu.sync_copy(pos_ref.at[pl.ds(lo, CL)], pos_buf)

            p = lax.rem(u, PPW)
            lo = pl.multiple_of(u - p, CL)
            _unit_norm(gbuf[s], obuf[s], pos_buf, seg_buf, p,
                       gam_buf, bet_buf, stats, rs_buf)

            @pl.when(u + 2 < NU)
            def _():
                pltpu.async_copy(
                    word_ref.at[ids_buf.at[pl.ds(_idx_off(u + 2), CL)]],
                    gbuf[s], gsem[s])

            pltpu.async_copy(obuf[s], out_ref.at[bb, n0 + p, pl.ds(lo, CL)],
                             ssem[s])
        return carry

    lax.fori_loop(0, NU // 2, step, 0)
    for s in range(2):
        pltpu.make_async_copy(pos_ref.at[pl.ds(0, CL)], obuf[s],
                              ssem[s]).wait()

    pltpu.sync_copy(tail_ref.at[pl.ds(w * PPW, PPW)], idx0)
    pltpu.async_copy(word_ref.at[idx0], gbuf0, gsem0).wait()
    pltpu.sync_copy(pos_ref.at[pl.ds(K, 8)], pos_buf.at[pl.ds(0, 8)])

    ktail = pl.multiple_of(jnp.int32(K) + 0 * n0, 8)

    def tail_row(p, carry):
        _norm_row(gbuf0, p, pos_buf, 0, seg_buf, p,
                  gam_buf, bet_buf, obuf0, 0)
        pltpu.sync_copy(obuf0.at[pl.ds(0, 8)],
                        out_ref.at[bb, n0 + p, pl.ds(ktail, 8)])
        return carry

    lax.fori_loop(0, PPW, tail_row, 0)


_sc_call = pl.kernel(
    _body,
    out_type=jax.ShapeDtypeStruct((B, N, L, H), jnp.float32),
    mesh=plsc.VectorSubcoreMesh(core_axis_name="c", subcore_axis_name="s"),
    compiler_params=pltpu.CompilerParams(needs_layout_passes=False),
    scratch_types=[
        pltpu.VMEM((CL,), jnp.int32),
        pltpu.VMEM((PPW * LP,), jnp.int32),
        pltpu.VMEM((CL, H), jnp.float32),
        pltpu.VMEM((CL, H), jnp.float32),
        pltpu.VMEM((CL, H), jnp.float32),
        pltpu.VMEM((CL, H), jnp.float32),
        pltpu.VMEM((CL, H), jnp.float32),
        pltpu.VMEM((PPW, H), jnp.float32),
        pltpu.VMEM((H,), jnp.float32),
        pltpu.VMEM((H,), jnp.float32),
        pltpu.VMEM((544,), jnp.float32),
        pltpu.VMEM((32,), jnp.float32),
        pltpu.SemaphoreType.DMA,
        pltpu.SemaphoreType.DMA,
        pltpu.SemaphoreType.DMA,
        pltpu.SemaphoreType.DMA,
    ],
)


@jax.jit
def kernel(input_ids, word_table, pos_table, seg_table, gamma, beta):
    ii = input_ids.astype(jnp.int32).reshape(PAIRS, K)
    ids1d = jnp.concatenate(
        [jnp.full((PAIRS, 1), CLS_SEG, jnp.int32), ii,
         jnp.zeros((PAIRS, LP - L), jnp.int32)], axis=1).reshape(-1)
    tail_ids = ii[:, K - 1]
    posp = jnp.pad(pos_table, ((0, 7), (0, 0)))
    return _sc_call(ids1d, tail_ids, word_table, posp, seg_table,
                    gamma, beta)


# speedup vs baseline: 1.3038x; 1.2820x over previous
import jax
import jax.numpy as jnp
from jax import lax
from jax.experimental import pallas as pl
from jax.experimental.pallas import tpu as pltpu
from jax.experimental.pallas import tpu_sc as plsc

B, N, K, H = 8, 64, 128, 768
V = 100000
L = K + 1
CLS_SEG = 1
EPS = 1e-12

NC, NS = 2, 16
NW = NC * NS
CL = 16
WPB = N // CL
NSL = H // 16
PCH = 16


def _rsqrt(x):
    xi = lax.bitcast_convert_type(x, jnp.int32)
    yi = jnp.full((16,), 0x5F3759DF, jnp.int32) - (xi >> 1)
    y = lax.bitcast_convert_type(yi, jnp.float32)
    for _ in range(3):
        y = y * (1.5 - 0.5 * x * y * y)
    return y


def _unit_norm(src, dst, seg_buf, pos_buf, pj, gam_buf, bet_buf, stats):
    lanes = lax.iota(jnp.int32, 16)

    def j_body(j, acc):
        sl = pl.ds(pl.multiple_of(j * 16, 16), 16)
        pos_j = pos_buf[pj, sl]
        out = []
        for i in range(16):
            x = src[i, sl] + seg_buf[i, sl] + pos_j
            dst[i, sl] = x
            out.append(acc[2 * i] + x)
            out.append(acc[2 * i + 1] + x * x)
        return tuple(out)

    acc0 = tuple(jnp.zeros((16,), jnp.float32) for _ in range(32))
    acc = lax.fori_loop(0, NSL, j_body, acc0)
    for i in range(16):
        plsc.store_scatter(stats, [lanes + 17 * i], acc[2 * i])
        plsc.store_scatter(stats, [lanes + (272 + 17 * i)], acc[2 * i + 1])
    tot_s = jnp.zeros((16,), jnp.float32)
    tot_q = jnp.zeros((16,), jnp.float32)
    for c in range(16):
        tot_s = tot_s + plsc.load_gather(stats, [lanes * 17 + c])
        tot_q = tot_q + plsc.load_gather(stats, [lanes * 17 + (272 + c)])
    mean = tot_s * (1.0 / H)
    msq = tot_q * (1.0 / H)
    rinv = _rsqrt(msq - mean * mean + EPS)
    shift = -mean * rinv

    for h in (0, 8):
        ri = [jnp.full((16,), rinv[i], jnp.float32) for i in range(h, h + 8)]
        sh = [jnp.full((16,), shift[i], jnp.float32) for i in range(h, h + 8)]

        def j2_body(j, carry, h=h, ri=ri, sh=sh):
            sl = pl.ds(pl.multiple_of(j * 16, 16), 16)
            g = gam_buf[sl]
            b = bet_buf[sl]
            for i in range(h, h + 8):
                dst[i, sl] = (dst[i, sl] * ri[i - h] + sh[i - h]) * g + b
            return carry

        lax.fori_loop(0, NSL, j2_body, 0)


def _body(ids_ref, word_ref, pos_ref, seg_ref, gamma_ref, beta_ref,
          out_ref,
          ids_buf, gbuf0, gbuf1, obuf0, obuf1,
          pos_buf, seg_buf, gam_buf, bet_buf, stats,
          gsem0, gsem1, ssem0, ssem1):
    w = lax.axis_index("s") * NC + lax.axis_index("c")
    bb = w // WPB
    n0 = (w % WPB) * CL
    gbuf = (gbuf0, gbuf1)
    obuf = (obuf0, obuf1)
    gsem = (gsem0, gsem1)
    ssem = (ssem0, ssem1)

    pltpu.sync_copy(gamma_ref, gam_buf)
    pltpu.sync_copy(beta_ref, bet_buf)
    pltpu.sync_copy(seg_ref.at[pl.ds(n0, CL)], seg_buf)
    pltpu.sync_copy(ids_ref.at[pl.ds(bb * (L * N), L * N)], ids_buf)

    def _idx(u):
        return ids_buf.at[pl.ds(pl.multiple_of(u * N + n0, 8), CL)]

    for s in range(2):
        pltpu.async_copy(word_ref.at[_idx(s)], gbuf[s], gsem[s])
        pltpu.async_copy(seg_buf, out_ref.at[bb, s, pl.ds(n0, CL)], ssem[s])

    def step(k, carry):
        for s in range(2):
            u = k * 2 + s
            pltpu.make_async_copy(
                pos_ref.at[pl.ds(0, CL)], obuf[s], ssem[s]).wait()
            pltpu.make_async_copy(
                word_ref.at[_idx(u)], gbuf[s], gsem[s]).wait()

            if s == 0:
                @pl.when(lax.rem(u, PCH) == 0)
                def _():
                    lo = pl.multiple_of(u, PCH)
                    pltpu.sync_copy(pos_ref.at[pl.ds(lo, PCH)], pos_buf)

            _unit_norm(gbuf[s], obuf[s], seg_buf, pos_buf, lax.rem(u, PCH),
                       gam_buf, bet_buf, stats)

            @pl.when(u + 2 < L)
            def _():
                pltpu.async_copy(word_ref.at[_idx(u + 2)], gbuf[s], gsem[s])

            pltpu.async_copy(obuf[s], out_ref.at[bb, u, pl.ds(n0, CL)],
                             ssem[s])
        return carry

    lax.fori_loop(0, K // 2, step, 0)

    pltpu.make_async_copy(pos_ref.at[pl.ds(0, CL)], obuf0, ssem0).wait()
    pltpu.make_async_copy(word_ref.at[_idx(K)], gbuf0, gsem0).wait()
    pltpu.sync_copy(pos_ref.at[pl.ds(K, 8)], pos_buf.at[pl.ds(0, 8)])
    _unit_norm(gbuf0, obuf0, seg_buf, pos_buf, 0, gam_buf, bet_buf, stats)
    pltpu.sync_copy(obuf0, out_ref.at[bb, K, pl.ds(n0, CL)])
    pltpu.make_async_copy(pos_ref.at[pl.ds(0, CL)], obuf1, ssem1).wait()


_sc_call = pl.kernel(
    _body,
    out_type=jax.ShapeDtypeStruct((B, L, N, H), jnp.float32),
    mesh=plsc.VectorSubcoreMesh(core_axis_name="c", subcore_axis_name="s"),
    compiler_params=pltpu.CompilerParams(needs_layout_passes=False),
    scratch_types=[
        pltpu.VMEM((L * N,), jnp.int32),
        pltpu.VMEM((CL, H), jnp.float32),
        pltpu.VMEM((CL, H), jnp.float32),
        pltpu.VMEM((CL, H), jnp.float32),
        pltpu.VMEM((CL, H), jnp.float32),
        pltpu.VMEM((PCH, H), jnp.float32),
        pltpu.VMEM((CL, H), jnp.float32),
        pltpu.VMEM((H,), jnp.float32),
        pltpu.VMEM((H,), jnp.float32),
        pltpu.VMEM((544,), jnp.float32),
        pltpu.SemaphoreType.DMA,
        pltpu.SemaphoreType.DMA,
        pltpu.SemaphoreType.DMA,
        pltpu.SemaphoreType.DMA,
    ],
)


@jax.jit
def kernel(input_ids, word_table, pos_table, seg_table, gamma, beta):
    ids = jnp.concatenate(
        [jnp.full((B, N, 1), CLS_SEG, dtype=input_ids.dtype), input_ids],
        axis=2)
    idsT = ids.astype(jnp.int32).transpose(0, 2, 1).reshape(-1)
    posp = jnp.pad(pos_table, ((0, 7), (0, 0)))
    out = _sc_call(idsT, word_table, posp, seg_table, gamma, beta)
    return out.transpose(0, 2, 1, 3)
